# Initial kernel scaffold; baseline (speedup 1.0000x reference)
import jax
import jax.numpy as jnp
from jax.experimental import pallas as pl
from jax.experimental.pallas import tpu as pltpu

_N = 10000
_G = 64
_D = 128


def _readout_body(bids_ref, x_ref, cw_ref, cb_ref, o_ref, acc_ref, cnt_ref):
    i = pl.program_id(0)

    @pl.when(i == 0)
    def _():
        acc_ref[...] = jnp.zeros_like(acc_ref)
        cnt_ref[...] = jnp.zeros_like(cnt_ref)

    bids = bids_ref[0, 0, :]
    rows = bids.shape[0]
    gi = jax.lax.broadcasted_iota(jnp.int32, (_G, rows), 0)
    onehot = (gi == bids[None, :]).astype(jnp.float32)
    acc_ref[...] += jnp.dot(onehot, x_ref[...], preferred_element_type=jnp.float32)
    cnt_ref[...] += jnp.sum(onehot, axis=1, keepdims=True)

    @pl.when(i == pl.num_programs(0) - 1)
    def _():
        emb = acc_ref[...] / jnp.clip(cnt_ref[...], 1.0)
        o_ref[...] = jnp.dot(emb, cw_ref[...].T, preferred_element_type=jnp.float32) + cb_ref[None, :]


def _readout(x, batch, cls_w, cls_b):
    nb = 25
    rows = _N // nb
    bids = batch.astype(jnp.int32).reshape(nb, 1, rows)
    return pl.pallas_call(
        _readout_body,
        grid=(nb,),
        in_specs=[
            pl.BlockSpec((1, 1, rows), lambda i: (i, 0, 0)),
            pl.BlockSpec((rows, _D), lambda i: (i, 0)),
            pl.BlockSpec((6, _D), lambda i: (0, 0)),
            pl.BlockSpec((6,), lambda i: (0,)),
        ],
        out_specs=pl.BlockSpec((_G, 6), lambda i: (0, 0)),
        out_shape=jax.ShapeDtypeStruct((_G, 6), jnp.float32),
        scratch_shapes=[
            pltpu.VMEM((_G, _D), jnp.float32),
            pltpu.VMEM((_G, 1), jnp.float32),
        ],
    )(bids, x, cls_w, cls_b)


def _layer(x, row, col, edge_attr, lw, lb, gw, gb):
    msg = edge_attr * x[row]
    summed = jax.ops.segment_sum(msg, col, num_segments=_N)
    cnt = jax.ops.segment_sum(jnp.ones((row.shape[0],), x.dtype), col, num_segments=_N)
    out = summed / jnp.clip(cnt, 1.0)[:, None]
    out = jax.nn.relu(out @ lw.T + lb)
    tau_hat = jax.nn.sigmoid(x @ gw.T + gb)
    tau_diff = jnp.abs(tau_hat[row] - tau_hat[col])
    tau_diff = tau_diff * edge_attr
    tau_sum = jnp.zeros_like(tau_hat).at[row].add(tau_diff)
    tau = jnp.tanh(tau_sum)
    return (1.0 - tau) * x + tau * out + x


def kernel(x, edge_index, edge_attr, batch,
           lin_w0, lin_b0, gate_w0, gate_b0,
           lin_w1, lin_b1, gate_w1, gate_b1,
           lin_w2, lin_b2, gate_w2, gate_b2,
           cls_w, cls_b):
    row, col = edge_index[0], edge_index[1]
    params = [
        (lin_w0, lin_b0, gate_w0, gate_b0),
        (lin_w1, lin_b1, gate_w1, gate_b1),
        (lin_w2, lin_b2, gate_w2, gate_b2),
    ]
    for lw, lb, gw, gb in params:
        x = _layer(x, row, col, edge_attr, lw, lb, gw, gb)
    return _readout(x, batch, cls_w, cls_b)


# speedup vs baseline: 3.4180x; 3.4180x over previous
import functools

import jax
import jax.numpy as jnp
from jax import lax
from jax.experimental import pallas as pl
from jax.experimental.pallas import tpu as pltpu
from jax.experimental.pallas import tpu_sc as plsc

_N = 10000
_E = 320000
_D = 128
_G = 64
_C = 128
_NCHUNK = _E // _C
_NW = 32
_NP = 10240
_RPS = _NP // 16
_ZR = 64

_mesh = plsc.VectorSubcoreMesh(core_axis_name="c", subcore_axis_name="s")


def _zero16():
    return jnp.zeros((16,), jnp.float32)


def _scale_rows_inplace(buf, eab, c):
    eav = eab[c, pl.ds(0, 16)]
    for r in range(8):
        buf[c, pl.ds(r * 16, 16)] = buf[c, pl.ds(r * 16, 16)] * eav


def _absdiff_rows_inplace(ga, gb, eab, c):
    eav = eab[c, pl.ds(0, 16)]
    for r in range(8):
        a = ga[c, pl.ds(r * 16, 16)]
        b = gb[c, pl.ds(r * 16, 16)]
        ga[c, pl.ds(r * 16, 16)] = jnp.abs(a - b) * eav


def _chunk_loop(wid, body, nchunk=_NCHUNK):
    nfull = nchunk // _NW
    rem = nchunk - nfull * _NW
    trips = nfull + jnp.where(wid < rem, 1, 0)

    def fbody(k, carry):
        body(wid + k * _NW)
        return carry

    lax.fori_loop(0, trips, fbody, 0)


def _zero_shared(zbuf, acc, sid, ncopy, zrows):
    @pl.loop(0, ncopy)
    def _(j):
        pltpu.sync_copy(zbuf, acc.at[pl.ds(sid * _RPS + j * zrows, zrows)])


@jax.jit
def _sc_cnt(col):
    out = jax.ShapeDtypeStruct((2, _NP, _D), jnp.float32)
    scratch = [
        pltpu.VMEM_SHARED((_NP, _D), jnp.float32),
        pltpu.VMEM((_C,), jnp.int32),
        pltpu.VMEM((_C, _D), jnp.float32),
    ]

    def body(col_h, cnt_h, cacc, cidx, ones):
        cid = lax.axis_index("c")
        sid = lax.axis_index("s")
        wid = sid * 2 + cid
        z16 = _zero16()
        o16 = jnp.ones((16,), jnp.float32)

        @pl.loop(0, _C)
        def _(i):
            for r in range(8):
                ones[i, pl.ds(r * 16, 16)] = z16

        _zero_shared(ones, cacc, sid, 5, _C)

        @pl.loop(0, _C)
        def _(i):
            for r in range(8):
                ones[i, pl.ds(r * 16, 16)] = o16

        plsc.subcore_barrier()

        def chunk(g):
            base = g * _C
            pltpu.sync_copy(col_h.at[pl.ds(base, _C)], cidx)
            pltpu.sync_copy(ones, cacc.at[cidx], add=True)

        _chunk_loop(wid, chunk)
        plsc.subcore_barrier()
        pltpu.sync_copy(cacc.at[pl.ds(sid * _RPS, _RPS)],
                        cnt_h.at[cid, pl.ds(sid * _RPS, _RPS)])

    k = pl.kernel(body, mesh=_mesh, out_type=out, scratch_types=scratch)
    return k(col)


@jax.jit
def _sc_phase_a(x, row, col, earep):
    out = jax.ShapeDtypeStruct((2, _NP, _D), jnp.float32)
    scratch = [
        pltpu.VMEM_SHARED((_NP, _D), jnp.float32),
        pltpu.VMEM((_C,), jnp.int32),
        pltpu.VMEM((_C,), jnp.int32),
        pltpu.VMEM((_C, 16), jnp.float32),
        pltpu.VMEM((_C, _D), jnp.float32),
        pltpu.SemaphoreType.DMA,
    ]

    def body(x_h, row_h, col_h, ea_h, out_h, acc, ridx, cidx, eab, gbuf, sem):
        cid = lax.axis_index("c")
        sid = lax.axis_index("s")
        wid = sid * 2 + cid
        z16 = _zero16()

        @pl.loop(0, _C)
        def _(i):
            for r in range(8):
                gbuf[i, pl.ds(r * 16, 16)] = z16

        _zero_shared(gbuf, acc, sid, 5, _C)
        plsc.subcore_barrier()

        def chunk(g):
            base = g * _C
            pltpu.sync_copy(row_h.at[pl.ds(base, _C)], ridx)
            pltpu.sync_copy(col_h.at[pl.ds(base, _C)], cidx)
            pltpu.sync_copy(ea_h.at[pl.ds(base, _C)], eab)
            pltpu.async_copy(x_h.at[ridx], gbuf, sem).wait()

            @pl.loop(0, _C)
            def _(c):
                _scale_rows_inplace(gbuf, eab, c)

            pltpu.sync_copy(gbuf, acc.at[cidx], add=True)

        _chunk_loop(wid, chunk)
        plsc.subcore_barrier()
        pltpu.sync_copy(acc.at[pl.ds(sid * _RPS, _RPS)],
                        out_h.at[cid, pl.ds(sid * _RPS, _RPS)])

    k = pl.kernel(body, mesh=_mesh, out_type=out, scratch_types=scratch)
    return k(x, row, col, earep)


_CB = 64
_NCHUNK_B = _E // _CB


@jax.jit
def _sc_phase_b(th, row, col, earep):
    out = jax.ShapeDtypeStruct((2, _NP, _D), jnp.float32)
    scratch = [
        pltpu.VMEM_SHARED((_NP, _D), jnp.float32),
        pltpu.VMEM((_CB,), jnp.int32),
        pltpu.VMEM((_CB,), jnp.int32),
        pltpu.VMEM((_CB, 16), jnp.float32),
        pltpu.VMEM((_CB, _D), jnp.float32),
        pltpu.VMEM((_CB, _D), jnp.float32),
        pltpu.SemaphoreType.DMA,
        pltpu.SemaphoreType.DMA,
    ]

    def body(th_h, row_h, col_h, ea_h, out_h, acc, ridx, cidx, eab, ga, gb,
             sema, semb):
        cid = lax.axis_index("c")
        sid = lax.axis_index("s")
        wid = sid * 2 + cid

        z16 = _zero16()

        @pl.loop(0, _CB)
        def _(i):
            for r in range(8):
                ga[i, pl.ds(r * 16, 16)] = z16

        _zero_shared(ga, acc, sid, 10, _CB)
        plsc.subcore_barrier()

        def chunk(g):
            base = g * _CB
            pltpu.sync_copy(row_h.at[pl.ds(base, _CB)], ridx)
            pltpu.sync_copy(col_h.at[pl.ds(base, _CB)], cidx)
            pltpu.sync_copy(ea_h.at[pl.ds(base, _CB)], eab)
            ca = pltpu.async_copy(th_h.at[ridx], ga, sema)
            cb = pltpu.async_copy(th_h.at[cidx], gb, semb)
            ca.wait()
            cb.wait()

            @pl.loop(0, _CB)
            def _(c):
                _absdiff_rows_inplace(ga, gb, eab, c)

            pltpu.sync_copy(ga, acc.at[ridx], add=True)

        _chunk_loop(wid, chunk, _NCHUNK_B)
        plsc.subcore_barrier()
        pltpu.sync_copy(acc.at[pl.ds(sid * _RPS, _RPS)],
                        out_h.at[cid, pl.ds(sid * _RPS, _RPS)])

    k = pl.kernel(body, mesh=_mesh, out_type=out, scratch_types=scratch)
    return k(th, row, col, earep)


_NB = 25
_BR = _N // _NB


def _tau0_body(x_ref, gw_ref, gb_ref, th_ref):
    th_ref[...] = jax.nn.sigmoid(
        jnp.dot(x_ref[...], gw_ref[...].T, preferred_element_type=jnp.float32)
        + gb_ref[...][None, :])


def _tau0(x, gw, gb):
    return pl.pallas_call(
        _tau0_body,
        grid=(_NB,),
        in_specs=[
            pl.BlockSpec((_BR, _D), lambda i: (i, 0)),
            pl.BlockSpec((_D, _D), lambda i: (0, 0)),
            pl.BlockSpec((_D,), lambda i: (0,)),
        ],
        out_specs=pl.BlockSpec((_BR, _D), lambda i: (i, 0)),
        out_shape=jax.ShapeDtypeStruct((_N, _D), jnp.float32),
    )(x, gw, gb)


def _combine_body(with_th, x_ref, pa_ref, pb_ref, cnt_ref, lw_ref, lb_ref,
                  gw_ref, gb_ref, xo_ref, *maybe_th):
    x = x_ref[...]
    pa = pa_ref[...]
    pb = pb_ref[...]
    cnt = cnt_ref[...]
    summed = pa[0] + pa[1]
    c = cnt[0][:, 0:1] + cnt[1][:, 0:1]
    aggr = summed / jnp.clip(c, 1.0)
    out = jnp.maximum(
        jnp.dot(aggr, lw_ref[...].T, preferred_element_type=jnp.float32)
        + lb_ref[...][None, :], 0.0)
    tau = jnp.tanh(pb[0] + pb[1])
    xn = (1.0 - tau) * x + tau * out + x
    xo_ref[...] = xn
    if with_th:
        maybe_th[0][...] = jax.nn.sigmoid(
            jnp.dot(xn, gw_ref[...].T, preferred_element_type=jnp.float32)
            + gb_ref[...][None, :])


def _combine(x, pa, pb, cnt, lw, lb, gw, gb, with_th):
    outs = [jax.ShapeDtypeStruct((_N, _D), jnp.float32)]
    out_specs = [pl.BlockSpec((_BR, _D), lambda i: (i, 0))]
    if with_th:
        outs.append(jax.ShapeDtypeStruct((_N, _D), jnp.float32))
        out_specs.append(pl.BlockSpec((_BR, _D), lambda i: (i, 0)))
    res = pl.pallas_call(
        functools.partial(_combine_body, with_th),
        grid=(_NB,),
        in_specs=[
            pl.BlockSpec((_BR, _D), lambda i: (i, 0)),
            pl.BlockSpec((2, _BR, _D), lambda i: (0, i, 0)),
            pl.BlockSpec((2, _BR, _D), lambda i: (0, i, 0)),
            pl.BlockSpec((2, _BR, _D), lambda i: (0, i, 0)),
            pl.BlockSpec((_D, _D), lambda i: (0, 0)),
            pl.BlockSpec((_D,), lambda i: (0,)),
            pl.BlockSpec((_D, _D), lambda i: (0, 0)),
            pl.BlockSpec((_D,), lambda i: (0,)),
        ],
        out_specs=out_specs,
        out_shape=outs,
    )(x, pa, pb, cnt, lw, lb, gw, gb)
    return res if with_th else (res[0], None)


def _readout_body(bids_ref, x_ref, cw_ref, cb_ref, o_ref, acc_ref, cnt_ref):
    i = pl.program_id(0)

    @pl.when(i == 0)
    def _():
        acc_ref[...] = jnp.zeros_like(acc_ref)
        cnt_ref[...] = jnp.zeros_like(cnt_ref)

    bids = bids_ref[0, 0, :]
    rows = bids.shape[0]
    gi = jax.lax.broadcasted_iota(jnp.int32, (_G, rows), 0)
    onehot = (gi == bids[None, :]).astype(jnp.float32)
    acc_ref[...] += jnp.dot(onehot, x_ref[...], preferred_element_type=jnp.float32)
    cnt_ref[...] += jnp.sum(onehot, axis=1, keepdims=True)

    @pl.when(i == pl.num_programs(0) - 1)
    def _():
        emb = acc_ref[...] / jnp.clip(cnt_ref[...], 1.0)
        o_ref[...] = jnp.dot(emb, cw_ref[...].T,
                             preferred_element_type=jnp.float32) + cb_ref[...][None, :]


def _readout(x, batch, cls_w, cls_b):
    bids = batch.astype(jnp.int32).reshape(_NB, 1, _BR)
    return pl.pallas_call(
        _readout_body,
        grid=(_NB,),
        in_specs=[
            pl.BlockSpec((1, 1, _BR), lambda i: (i, 0, 0)),
            pl.BlockSpec((_BR, _D), lambda i: (i, 0)),
            pl.BlockSpec((6, _D), lambda i: (0, 0)),
            pl.BlockSpec((6,), lambda i: (0,)),
        ],
        out_specs=pl.BlockSpec((_G, 6), lambda i: (0, 0)),
        out_shape=jax.ShapeDtypeStruct((_G, 6), jnp.float32),
        scratch_shapes=[
            pltpu.VMEM((_G, _D), jnp.float32),
            pltpu.VMEM((_G, 1), jnp.float32),
        ],
    )(bids, x, cls_w, cls_b)


def kernel(x, edge_index, edge_attr, batch,
           lin_w0, lin_b0, gate_w0, gate_b0,
           lin_w1, lin_b1, gate_w1, gate_b1,
           lin_w2, lin_b2, gate_w2, gate_b2,
           cls_w, cls_b):
    row = edge_index[0].astype(jnp.int32)
    col = edge_index[1].astype(jnp.int32)
    earep = jnp.broadcast_to(edge_attr.astype(jnp.float32), (_E, 16))
    params = [
        (lin_w0, lin_b0, gate_w0, gate_b0),
        (lin_w1, lin_b1, gate_w1, gate_b1),
        (lin_w2, lin_b2, gate_w2, gate_b2),
    ]
    th = _tau0(x, gate_w0, gate_b0)
    cnt = _sc_cnt(col)
    for i, (lw, lb, gw, gb) in enumerate(params):
        pa = _sc_phase_a(x, row, col, earep)
        pb = _sc_phase_b(th, row, col, earep)
        with_th = i < 2
        ngw, ngb = (params[i + 1][2], params[i + 1][3]) if with_th else (gw, gb)
        x, th = _combine(x, pa, pb, cnt, lw, lb, ngw, ngb, with_th)
    return _readout(x, batch, cls_w, cls_b)
